# baseline (speedup 1.0000x reference)
import functools

import jax
import jax.numpy as jnp
from jax.experimental import pallas as pl
from jax.experimental.pallas import tpu as pltpu

EPS = 1e-5
D1 = 384
D2 = 256
NSEL = 128


def _pack(params):
    pw, pc, ps, pt = (params[k] for k in ("women", "children", "sc", "st"))
    w1 = jnp.concatenate([pw["W1"], pc["W1"], ps["W1"], pt["W1"]], axis=1)
    b1 = jnp.concatenate([pw["b1"], pc["b1"], ps["b1"], pt["b1"]])[None, :]
    g1 = jnp.concatenate([pw["g1"], pc["g1"], ps["g1"], pt["g1"]])[None, :]
    be1 = jnp.concatenate([pw["be1"], pc["be1"], ps["be1"], pt["be1"]])[None, :]

    w2 = jnp.zeros((D1, D2), jnp.float32)
    w2 = w2.at[0:128, 0:64].set(pw["W2"])
    w2 = w2.at[128:256, 64:128].set(pc["W2"])
    w2 = w2.at[256:320, 128:160].set(ps["W2"])
    w2 = w2.at[320:384, 160:192].set(pt["W2"])
    z64 = jnp.zeros((64,), jnp.float32)
    b2 = jnp.concatenate([pw["b2"], pc["b2"], ps["b2"], pt["b2"], z64])[None, :]
    g2 = jnp.concatenate([pw["g2"], pc["g2"], ps["g2"], pt["g2"], z64])[None, :]
    be2 = jnp.concatenate([pw["be2"], pc["be2"], ps["be2"], pt["be2"], z64])[None, :]

    w3 = jnp.zeros((D2, NSEL), jnp.float32)
    w3 = w3.at[0:64, 2].set(pw["W3"][:, 0])
    w3 = w3.at[64:128, 3].set(pc["W3"][:, 0])
    w3 = w3.at[128:160, 0].set(ps["W3"][:, 0])
    w3 = w3.at[160:192, 1].set(pt["W3"][:, 0])
    b3 = jnp.zeros((1, NSEL), jnp.float32)
    b3 = b3.at[0, 0].set(ps["b3"][0]).at[0, 1].set(pt["b3"][0])
    b3 = b3.at[0, 2].set(pw["b3"][0]).at[0, 3].set(pc["b3"][0])

    return (w1.astype(jnp.bfloat16), b1, g1, be1,
            w2.astype(jnp.bfloat16), b2, g2, be2,
            w3.astype(jnp.bfloat16), b3)


def _ln_full(h, g, be):
    mu = jnp.mean(h, axis=-1, keepdims=True)
    var = jnp.mean(h * h, axis=-1, keepdims=True) - mu * mu
    return (h - mu) * jax.lax.rsqrt(var + EPS) * g + be


def _ln_2seg(h, s, g, be):
    lanes = jax.lax.broadcasted_iota(jnp.int32, h.shape, 1)
    in_a = lanes < s
    in_ab = lanes < 2 * s
    zero = jnp.zeros_like(h)
    ha = jnp.where(in_a, h, zero)
    hab = jnp.where(in_ab, h, zero)
    sa = jnp.sum(ha, axis=-1, keepdims=True)
    sab = jnp.sum(hab, axis=-1, keepdims=True)
    qa = jnp.sum(ha * ha, axis=-1, keepdims=True)
    qab = jnp.sum(hab * hab, axis=-1, keepdims=True)
    inv = 1.0 / s
    mua, mub = sa * inv, (sab - sa) * inv
    vara = qa * inv - mua * mua
    varb = (qab - qa) * inv - mub * mub
    rsa = jax.lax.rsqrt(vara + EPS)
    rsb = jax.lax.rsqrt(varb + EPS)
    mu = jnp.where(in_a, mua, jnp.where(in_ab, mub, zero))
    rstd = jnp.where(in_a, rsa, rsb)
    return (h - mu) * rstd * g + be


def _body(x_ref, lab_ref, w1_ref, b1_ref, g1_ref, be1_ref,
          w2_ref, b2_ref, g2_ref, be2_ref, w3_ref, b3_ref, o_ref):
    dot = functools.partial(jax.lax.dot_general,
                            dimension_numbers=(((1,), (0,)), ((), ())),
                            preferred_element_type=jnp.float32)
    xb = x_ref[...].astype(jnp.bfloat16)
    h = dot(xb, w1_ref[...]) + b1_ref[...]
    g1, be1 = g1_ref[...], be1_ref[...]
    ln0 = _ln_full(h[:, 0:128], g1[:, 0:128], be1[:, 0:128])
    ln1 = _ln_full(h[:, 128:256], g1[:, 128:256], be1[:, 128:256])
    ln2 = _ln_2seg(h[:, 256:384], 64, g1[:, 256:384], be1[:, 256:384])
    hn = jnp.maximum(jnp.concatenate([ln0, ln1, ln2], axis=1), 0.0)

    h2 = dot(hn.astype(jnp.bfloat16), w2_ref[...]) + b2_ref[...]
    g2, be2 = g2_ref[...], be2_ref[...]
    m0 = _ln_2seg(h2[:, 0:128], 64, g2[:, 0:128], be2[:, 0:128])
    m1 = _ln_2seg(h2[:, 128:256], 32, g2[:, 128:256], be2[:, 128:256])
    hn2 = jnp.maximum(jnp.concatenate([m0, m1], axis=1), 0.0)

    p = dot(hn2.astype(jnp.bfloat16), w3_ref[...]) + b3_ref[...]
    lab = lab_ref[...]
    lanes = jax.lax.broadcasted_iota(jnp.int32, p.shape, 1)
    o_ref[...] = jnp.sum(jnp.where(lanes == lab, p, 0.0),
                         axis=1, keepdims=True)


def kernel(x, group_labels, params):
    n, d = x.shape
    blk = 1024
    labels = group_labels.astype(jnp.int32).reshape(n, 1)
    packed = _pack(params)

    const = lambda a: pl.BlockSpec(a.shape, lambda i: (0, 0))
    return pl.pallas_call(
        _body,
        grid=(n // blk,),
        in_specs=[
            pl.BlockSpec((blk, d), lambda i: (i, 0)),
            pl.BlockSpec((blk, 1), lambda i: (i, 0)),
        ] + [const(a) for a in packed],
        out_specs=pl.BlockSpec((blk, 1), lambda i: (i, 0)),
        out_shape=jax.ShapeDtypeStruct((n, 1), x.dtype),
        compiler_params=pltpu.CompilerParams(
            dimension_semantics=("arbitrary",)),
    )(x, labels, *packed)


# speedup vs baseline: 1.3184x; 1.3184x over previous
import functools

import jax
import jax.numpy as jnp
from jax.experimental import pallas as pl
from jax.experimental.pallas import tpu as pltpu

EPS = 1e-5
GROUP_ORDER = ("women", "children", "sc", "st")
PKEYS = ("W1", "b1", "g1", "be1", "W2", "b2", "g2", "be2", "W3", "b3")


def _ln_full(h, g, be):
    mu = jnp.mean(h, axis=-1, keepdims=True)
    var = jnp.mean(h * h, axis=-1, keepdims=True) - mu * mu
    return (h - mu) * jax.lax.rsqrt(var + EPS) * g + be


def _ln_2seg(h, s, g, be):
    lanes = jax.lax.broadcasted_iota(jnp.int32, h.shape, 1)
    in_a = lanes < s
    in_ab = lanes < 2 * s
    zero = jnp.zeros_like(h)
    ha = jnp.where(in_a, h, zero)
    hab = jnp.where(in_ab, h, zero)
    sa = jnp.sum(ha, axis=-1, keepdims=True)
    sab = jnp.sum(hab, axis=-1, keepdims=True)
    qa = jnp.sum(ha * ha, axis=-1, keepdims=True)
    qab = jnp.sum(hab * hab, axis=-1, keepdims=True)
    inv = 1.0 / s
    mua, mub = sa * inv, (sab - sa) * inv
    vara = qa * inv - mua * mua
    varb = (qab - qa) * inv - mub * mub
    rsa = jax.lax.rsqrt(vara + EPS)
    rsb = jax.lax.rsqrt(varb + EPS)
    mu = jnp.where(in_a, mua, jnp.where(in_ab, mub, zero))
    rstd = jnp.where(in_a, rsa, rsb)
    return (h - mu) * rstd * g + be


def _bf(a):
    return a.astype(jnp.bfloat16)


def _body(x_ref, lab_ref, *refs):
    (w1w, b1w, g1w, be1w, w2w, b2w, g2w, be2w, w3w, b3w,
     w1c, b1c, g1c, be1c, w2c, b2c, g2c, be2c, w3c, b3c,
     w1s, b1s, g1s, be1s, w2s, b2s, g2s, be2s, w3s, b3s,
     w1t, b1t, g1t, be1t, w2t, b2t, g2t, be2t, w3t, b3t,
     o_ref) = refs
    dot = functools.partial(jax.lax.dot_general,
                            dimension_numbers=(((1,), (0,)), ((), ())),
                            preferred_element_type=jnp.float32)
    f32 = jnp.float32
    xb = _bf(x_ref[...])

    h0 = dot(xb, _bf(w1w[...])) + b1w[...]
    h1 = dot(xb, _bf(w1c[...])) + b1c[...]
    w1st = jnp.concatenate([w1s[...], w1t[...]], axis=1)
    b1st = jnp.concatenate([b1s[...], b1t[...]])
    h2 = dot(xb, _bf(w1st)) + b1st

    ln0 = jnp.maximum(_ln_full(h0, g1w[...], be1w[...]), 0.0)
    ln1 = jnp.maximum(_ln_full(h1, g1c[...], be1c[...]), 0.0)
    g1st = jnp.concatenate([g1s[...], g1t[...]])
    be1st = jnp.concatenate([be1s[...], be1t[...]])
    ln2 = jnp.maximum(_ln_2seg(h2, 64, g1st, be1st), 0.0)

    z64_64 = jnp.zeros((64, 64), f32)
    w2w_ext = jnp.concatenate([w2w[...], jnp.zeros((128, 64), f32)], axis=1)
    w2c_ext = jnp.concatenate([jnp.zeros((128, 64), f32), w2c[...]], axis=1)
    ha = dot(_bf(ln0), _bf(w2w_ext)) + dot(_bf(ln1), _bf(w2c_ext))
    b2a = jnp.concatenate([b2w[...], b2c[...]])
    ha = ha + b2a

    top = jnp.concatenate([w2s[...], jnp.zeros((64, 32), f32), z64_64], axis=1)
    bot = jnp.concatenate([jnp.zeros((64, 32), f32), w2t[...], z64_64], axis=1)
    w2st_ext = jnp.concatenate([top, bot], axis=0)
    z64 = jnp.zeros((64,), f32)
    b2b = jnp.concatenate([b2s[...], b2t[...], z64])
    hb = dot(_bf(ln2), _bf(w2st_ext)) + b2b

    g2a = jnp.concatenate([g2w[...], g2c[...]])
    be2a = jnp.concatenate([be2w[...], be2c[...]])
    lna = jnp.maximum(_ln_2seg(ha, 64, g2a, be2a), 0.0)
    g2b = jnp.concatenate([g2s[...], g2t[...], z64])
    be2b = jnp.concatenate([be2s[...], be2t[...], z64])
    lnb = jnp.maximum(_ln_2seg(hb, 32, g2b, be2b), 0.0)

    z64_1 = jnp.zeros((64, 1), f32)
    z32_1 = jnp.zeros((32, 1), f32)
    col2 = jnp.concatenate([w3w[...], z64_1], axis=0)
    col3 = jnp.concatenate([z64_1, w3c[...]], axis=0)
    w3a = jnp.concatenate(
        [jnp.zeros((128, 2), f32), col2, col3, jnp.zeros((128, 124), f32)],
        axis=1)
    col0 = jnp.concatenate([w3s[...], z32_1, z64_1], axis=0)
    col1 = jnp.concatenate([z32_1, w3t[...], z64_1], axis=0)
    w3b = jnp.concatenate([col0, col1, jnp.zeros((128, 126), f32)], axis=1)
    b3vec = jnp.concatenate(
        [b3s[...], b3t[...], b3w[...], b3c[...], jnp.zeros((124,), f32)])
    p = dot(_bf(lna), _bf(w3a)) + dot(_bf(lnb), _bf(w3b)) + b3vec

    lab = lab_ref[...]
    lanes = jax.lax.broadcasted_iota(jnp.int32, p.shape, 1)
    o_ref[...] = jnp.sum(jnp.where(lanes == lab, p, 0.0),
                         axis=1, keepdims=True)


def kernel(x, group_labels, params):
    n, d = x.shape
    blk = 1024
    labels = group_labels.astype(jnp.int32).reshape(n, 1)

    raw = []
    specs = [
        pl.BlockSpec((blk, d), lambda i: (i, 0)),
        pl.BlockSpec((blk, 1), lambda i: (i, 0)),
    ]
    for name in GROUP_ORDER:
        for k in PKEYS:
            a = params[name][k]
            raw.append(a)
            if a.ndim == 1:
                specs.append(pl.BlockSpec(a.shape, lambda i: (0,)))
            else:
                specs.append(pl.BlockSpec(a.shape, lambda i: (0, 0)))

    return pl.pallas_call(
        _body,
        grid=(n // blk,),
        in_specs=specs,
        out_specs=pl.BlockSpec((blk, 1), lambda i: (i, 0)),
        out_shape=jax.ShapeDtypeStruct((n, 1), x.dtype),
        compiler_params=pltpu.CompilerParams(
            dimension_semantics=("arbitrary",)),
    )(x, labels, *raw)


# speedup vs baseline: 1.4313x; 1.0857x over previous
import functools

import jax
import jax.numpy as jnp
from jax.experimental import pallas as pl
from jax.experimental.pallas import tpu as pltpu

EPS = 1e-5

_OFF_B1, _OFF_G1, _OFF_BE1 = 0, 384, 768
_OFF_B2A, _OFF_B2B = 1152, 1280
_OFF_G2A, _OFF_G2B = 1408, 1536
_OFF_BE2A, _OFF_BE2B = 1664, 1792
_OFF_W3A, _OFF_W3B, _OFF_B3 = 1920, 2048, 2176
_VEC_LEN = 2304


def _ln_full(h, g, be):
    mu = jnp.mean(h, axis=-1, keepdims=True)
    var = jnp.mean(h * h, axis=-1, keepdims=True) - mu * mu
    return (h - mu) * jax.lax.rsqrt(var + EPS) * g + be


def _ln_2seg(h, s, g, be):
    lanes = jax.lax.broadcasted_iota(jnp.int32, h.shape, 1)
    in_a = lanes < s
    in_ab = lanes < 2 * s
    zero = jnp.zeros_like(h)
    ha = jnp.where(in_a, h, zero)
    hab = jnp.where(in_ab, h, zero)
    sa = jnp.sum(ha, axis=-1, keepdims=True)
    sab = jnp.sum(hab, axis=-1, keepdims=True)
    qa = jnp.sum(ha * ha, axis=-1, keepdims=True)
    qab = jnp.sum(hab * hab, axis=-1, keepdims=True)
    inv = 1.0 / s
    mua, mub = sa * inv, (sab - sa) * inv
    vara = qa * inv - mua * mua
    varb = (qab - qa) * inv - mub * mub
    rsa = jax.lax.rsqrt(vara + EPS)
    rsb = jax.lax.rsqrt(varb + EPS)
    mu = jnp.where(in_a, mua, jnp.where(in_ab, mub, zero))
    rstd = jnp.where(in_a, rsa, rsb)
    return (h - mu) * rstd * g + be


def _bf(a):
    return a.astype(jnp.bfloat16)


def _body(x_ref, lab_ref, w1_ref, w2wc_ref, w2st_ref, vec_ref, o_ref):
    dot = functools.partial(jax.lax.dot_general,
                            dimension_numbers=(((1,), (0,)), ((), ())),
                            preferred_element_type=jnp.float32)
    vec = lambda off, ln: vec_ref[pl.ds(off, ln)]
    xb = _bf(x_ref[...])

    h = dot(xb, _bf(w1_ref[...])) + vec(_OFF_B1, 384)
    g1 = vec(_OFF_G1, 384)
    be1 = vec(_OFF_BE1, 384)
    ln0 = jnp.maximum(
        _ln_full(h[:, 0:128], g1[0:128], be1[0:128]), 0.0)
    ln1 = jnp.maximum(
        _ln_full(h[:, 128:256], g1[128:256], be1[128:256]), 0.0)
    ln2 = jnp.maximum(
        _ln_2seg(h[:, 256:384], 64, g1[256:384], be1[256:384]), 0.0)

    w2wc = w2wc_ref[...]
    wlanes = jax.lax.broadcasted_iota(jnp.int32, w2wc.shape, 1)
    w2a = jnp.concatenate([jnp.where(wlanes < 64, w2wc, 0.0),
                           jnp.where(wlanes >= 64, w2wc, 0.0)], axis=0)
    ln01 = jnp.concatenate([ln0, ln1], axis=1)
    ha = dot(_bf(ln01), _bf(w2a)) + vec(_OFF_B2A, 128)

    w2st = w2st_ref[...]
    slanes = jax.lax.broadcasted_iota(jnp.int32, w2st.shape, 1)
    w2b = jnp.concatenate([jnp.where(slanes < 32, w2st, 0.0),
                           jnp.where((slanes >= 32) & (slanes < 64), w2st, 0.0)],
                          axis=0)
    hb = dot(_bf(ln2), _bf(w2b)) + vec(_OFF_B2B, 128)

    lna = jnp.maximum(
        _ln_2seg(ha, 64, vec(_OFF_G2A, 128), vec(_OFF_BE2A, 128)), 0.0)
    lnb = jnp.maximum(
        _ln_2seg(hb, 32, vec(_OFF_G2B, 128), vec(_OFF_BE2B, 128)), 0.0)

    q = lna * vec(_OFF_W3A, 128)
    r = lnb * vec(_OFF_W3B, 128)
    lanes = jax.lax.broadcasted_iota(jnp.int32, q.shape, 1)
    zero = jnp.zeros_like(q)
    qw = jnp.sum(jnp.where(lanes < 64, q, zero), axis=1, keepdims=True)
    qfull = jnp.sum(q, axis=1, keepdims=True)
    rs = jnp.sum(jnp.where(lanes < 32, r, zero), axis=1, keepdims=True)
    r64 = jnp.sum(jnp.where(lanes < 64, r, zero), axis=1, keepdims=True)

    b3 = vec(_OFF_B3, 128)
    lab = lab_ref[...]
    p_sc = rs + b3[0]
    p_st = (r64 - rs) + b3[1]
    p_w = qw + b3[2]
    p_c = (qfull - qw) + b3[3]
    preds = jnp.where(lab < 2,
                      jnp.where(lab == 0, p_sc, p_st),
                      jnp.where(lab == 2, p_w, p_c))
    o_ref[...] = preds


def kernel(x, group_labels, params):
    n, d = x.shape
    blk = 1024
    labels = group_labels.astype(jnp.int32).reshape(n, 1)
    pw, pc, ps, pt = (params[k] for k in ("women", "children", "sc", "st"))

    w1 = jnp.concatenate([pw["W1"], pc["W1"], ps["W1"], pt["W1"]], axis=1)
    w2wc = jnp.concatenate([pw["W2"], pc["W2"]], axis=1)
    z = jnp.zeros((64, 64), jnp.float32)
    w2st = jnp.concatenate([ps["W2"], pt["W2"], z], axis=1)
    z64 = jnp.zeros((64,), jnp.float32)
    vecs = jnp.concatenate([
        pw["b1"], pc["b1"], ps["b1"], pt["b1"],
        pw["g1"], pc["g1"], ps["g1"], pt["g1"],
        pw["be1"], pc["be1"], ps["be1"], pt["be1"],
        pw["b2"], pc["b2"], ps["b2"], pt["b2"], z64,
        pw["g2"], pc["g2"], ps["g2"], pt["g2"], z64,
        pw["be2"], pc["be2"], ps["be2"], pt["be2"], z64,
        pw["W3"][:, 0], pc["W3"][:, 0],
        ps["W3"][:, 0], pt["W3"][:, 0], z64,
        ps["b3"], pt["b3"], pw["b3"], pc["b3"],
        jnp.zeros((124,), jnp.float32),
    ])
    assert vecs.shape == (_VEC_LEN,)

    return pl.pallas_call(
        _body,
        grid=(n // blk,),
        in_specs=[
            pl.BlockSpec((blk, d), lambda i: (i, 0)),
            pl.BlockSpec((blk, 1), lambda i: (i, 0)),
            pl.BlockSpec(w1.shape, lambda i: (0, 0)),
            pl.BlockSpec(w2wc.shape, lambda i: (0, 0)),
            pl.BlockSpec(w2st.shape, lambda i: (0, 0)),
            pl.BlockSpec(vecs.shape, lambda i: (0,)),
        ],
        out_specs=pl.BlockSpec((blk, 1), lambda i: (i, 0)),
        out_shape=jax.ShapeDtypeStruct((n, 1), x.dtype),
        compiler_params=pltpu.CompilerParams(
            dimension_semantics=("arbitrary",)),
    )(x, labels, w1, w2wc, w2st, vecs)


# speedup vs baseline: 2.4357x; 1.7017x over previous
import functools

import jax
import jax.numpy as jnp
from jax.experimental import pallas as pl
from jax.experimental.pallas import tpu as pltpu

EPS = 1e-5

_OFF_B1, _OFF_G1, _OFF_BE1 = 0, 384, 768
_OFF_B2A, _OFF_B2B = 1152, 1280
_OFF_G2A, _OFF_G2B = 1408, 1536
_OFF_BE2A, _OFF_BE2B = 1664, 1792
_OFF_W3A, _OFF_W3B, _OFF_B3 = 1920, 2048, 2176
_VEC_LEN = 2304


def _lnt(h, g, be):
    mu = jnp.mean(h, axis=0, keepdims=True)
    var = jnp.mean(h * h, axis=0, keepdims=True) - mu * mu
    return jnp.maximum((h - mu) * jax.lax.rsqrt(var + EPS) * g + be, 0.0)


def _bf(a):
    return a.astype(jnp.bfloat16)


def _body(x_ref, lab_ref, w1_ref, w2wc_ref, w2st_ref, vec_ref, o_ref):
    vec = lambda off, ln: vec_ref[pl.ds(off, ln), :]
    xb = _bf(x_ref[...])

    ht = jax.lax.dot_general(
        _bf(w1_ref[...]), xb, (((0,), (1,)), ((), ())),
        preferred_element_type=jnp.float32)
    ht = ht + vec(_OFF_B1, 384)
    g1 = vec(_OFF_G1, 384)
    be1 = vec(_OFF_BE1, 384)
    ln0 = _lnt(ht[0:128], g1[0:128], be1[0:128])
    ln1 = _lnt(ht[128:256], g1[128:256], be1[128:256])
    lns = _lnt(ht[256:320], g1[256:320], be1[256:320])
    lnt_ = _lnt(ht[320:384], g1[320:384], be1[320:384])

    w2wc = w2wc_ref[...]
    wlanes = jax.lax.broadcasted_iota(jnp.int32, w2wc.shape, 1)
    w2a = jnp.concatenate([jnp.where(wlanes < 64, w2wc, 0.0),
                           jnp.where(wlanes >= 64, w2wc, 0.0)], axis=0)
    ln01 = jnp.concatenate([ln0, ln1], axis=0)
    hat = jax.lax.dot_general(
        _bf(w2a), _bf(ln01), (((0,), (0,)), ((), ())),
        preferred_element_type=jnp.float32)
    hat = hat + vec(_OFF_B2A, 128)

    w2st = w2st_ref[...]
    slanes = jax.lax.broadcasted_iota(jnp.int32, w2st.shape, 1)
    w2b = jnp.concatenate([jnp.where(slanes < 32, w2st, 0.0),
                           jnp.where((slanes >= 32) & (slanes < 64), w2st, 0.0)],
                          axis=0)
    ln23 = jnp.concatenate([lns, lnt_], axis=0)
    hbt = jax.lax.dot_general(
        _bf(w2b), _bf(ln23), (((0,), (0,)), ((), ())),
        preferred_element_type=jnp.float32)
    hbt = hbt + vec(_OFF_B2B, 128)

    g2a, be2a = vec(_OFF_G2A, 128), vec(_OFF_BE2A, 128)
    g2b, be2b = vec(_OFF_G2B, 128), vec(_OFF_BE2B, 128)
    lnw2 = _lnt(hat[0:64], g2a[0:64], be2a[0:64])
    lnc2 = _lnt(hat[64:128], g2a[64:128], be2a[64:128])
    lns2 = _lnt(hbt[0:32], g2b[0:32], be2b[0:32])
    lnt2 = _lnt(hbt[32:64], g2b[32:64], be2b[32:64])

    w3a = vec(_OFF_W3A, 128)
    w3b = vec(_OFF_W3B, 128)
    p_w = jnp.sum(lnw2 * w3a[0:64], axis=0, keepdims=True)
    p_c = jnp.sum(lnc2 * w3a[64:128], axis=0, keepdims=True)
    p_s = jnp.sum(lns2 * w3b[0:32], axis=0, keepdims=True)
    p_t = jnp.sum(lnt2 * w3b[32:64], axis=0, keepdims=True)

    lab = lab_ref[...]
    preds = jnp.where(
        lab < 2,
        jnp.where(lab == 0, p_s + vec_ref[_OFF_B3, 0],
                  p_t + vec_ref[_OFF_B3 + 1, 0]),
        jnp.where(lab == 2, p_w + vec_ref[_OFF_B3 + 2, 0],
                  p_c + vec_ref[_OFF_B3 + 3, 0]))
    o_ref[...] = preds


def kernel(x, group_labels, params):
    n, d = x.shape
    blk = 1024
    labels = group_labels.astype(jnp.int32).reshape(1, n)
    pw, pc, ps, pt = (params[k] for k in ("women", "children", "sc", "st"))

    w1 = jnp.concatenate([pw["W1"], pc["W1"], ps["W1"], pt["W1"]], axis=1)
    w2wc = jnp.concatenate([pw["W2"], pc["W2"]], axis=1)
    z = jnp.zeros((64, 64), jnp.float32)
    w2st = jnp.concatenate([ps["W2"], pt["W2"], z], axis=1)
    z64 = jnp.zeros((64,), jnp.float32)
    vecs = jnp.concatenate([
        pw["b1"], pc["b1"], ps["b1"], pt["b1"],
        pw["g1"], pc["g1"], ps["g1"], pt["g1"],
        pw["be1"], pc["be1"], ps["be1"], pt["be1"],
        pw["b2"], pc["b2"], ps["b2"], pt["b2"], z64,
        pw["g2"], pc["g2"], ps["g2"], pt["g2"], z64,
        pw["be2"], pc["be2"], ps["be2"], pt["be2"], z64,
        pw["W3"][:, 0], pc["W3"][:, 0],
        ps["W3"][:, 0], pt["W3"][:, 0], z64,
        ps["b3"], pt["b3"], pw["b3"], pc["b3"],
        jnp.zeros((124,), jnp.float32),
    ])[:, None]

    out = pl.pallas_call(
        _body,
        grid=(n // blk,),
        in_specs=[
            pl.BlockSpec((blk, d), lambda i: (i, 0)),
            pl.BlockSpec((1, blk), lambda i: (0, i)),
            pl.BlockSpec(w1.shape, lambda i: (0, 0)),
            pl.BlockSpec(w2wc.shape, lambda i: (0, 0)),
            pl.BlockSpec(w2st.shape, lambda i: (0, 0)),
            pl.BlockSpec(vecs.shape, lambda i: (0, 0)),
        ],
        out_specs=pl.BlockSpec((1, blk), lambda i: (0, i)),
        out_shape=jax.ShapeDtypeStruct((1, n), x.dtype),
        compiler_params=pltpu.CompilerParams(
            dimension_semantics=("arbitrary",)),
    )(x, labels, w1, w2wc, w2st, vecs)
    return out.reshape(n, 1)


# speedup vs baseline: 2.5572x; 1.0499x over previous
import functools

import jax
import jax.numpy as jnp
from jax.experimental import pallas as pl
from jax.experimental.pallas import tpu as pltpu

EPS = 1e-5

_OFF_B1, _OFF_G1, _OFF_BE1 = 0, 384, 768
_OFF_B2A, _OFF_B2B = 1152, 1280
_OFF_G2A, _OFF_G2B = 1408, 1536
_OFF_BE2A, _OFF_BE2B = 1664, 1792
_OFF_W3A, _OFF_W3B, _OFF_B3 = 1920, 2048, 2176
_VEC_LEN = 2304


def _lnt(h, g, be):
    mu = jnp.mean(h, axis=0, keepdims=True)
    var = jnp.mean(h * h, axis=0, keepdims=True) - mu * mu
    return jnp.maximum((h - mu) * jax.lax.rsqrt(var + EPS) * g + be, 0.0)


def _bf(a):
    return a.astype(jnp.bfloat16)


def _body(x_ref, lab_ref, w1_ref, w2wc_ref, w2st_ref, vec_ref, o_ref):
    vec = lambda off, ln: vec_ref[pl.ds(off, ln), :]
    xb = _bf(x_ref[...])

    ht = jax.lax.dot_general(
        _bf(w1_ref[...]), xb, (((0,), (1,)), ((), ())),
        preferred_element_type=jnp.float32)
    ht = ht + vec(_OFF_B1, 384)
    g1 = vec(_OFF_G1, 384)
    be1 = vec(_OFF_BE1, 384)
    ln0 = _lnt(ht[0:128], g1[0:128], be1[0:128])
    ln1 = _lnt(ht[128:256], g1[128:256], be1[128:256])
    lns = _lnt(ht[256:320], g1[256:320], be1[256:320])
    lnt_ = _lnt(ht[320:384], g1[320:384], be1[320:384])

    w2wc = w2wc_ref[...]
    wlanes = jax.lax.broadcasted_iota(jnp.int32, w2wc.shape, 1)
    w2a = jnp.concatenate([jnp.where(wlanes < 64, w2wc, 0.0),
                           jnp.where(wlanes >= 64, w2wc, 0.0)], axis=0)
    ln01 = jnp.concatenate([ln0, ln1], axis=0)
    hat = jax.lax.dot_general(
        _bf(w2a), _bf(ln01), (((0,), (0,)), ((), ())),
        preferred_element_type=jnp.float32)
    hat = hat + vec(_OFF_B2A, 128)

    w2st = w2st_ref[...]
    slanes = jax.lax.broadcasted_iota(jnp.int32, w2st.shape, 1)
    w2b = jnp.concatenate([jnp.where(slanes < 32, w2st, 0.0),
                           jnp.where((slanes >= 32) & (slanes < 64), w2st, 0.0)],
                          axis=0)
    ln23 = jnp.concatenate([lns, lnt_], axis=0)
    hbt = jax.lax.dot_general(
        _bf(w2b), _bf(ln23), (((0,), (0,)), ((), ())),
        preferred_element_type=jnp.float32)
    hbt = hbt + vec(_OFF_B2B, 128)

    g2a, be2a = vec(_OFF_G2A, 128), vec(_OFF_BE2A, 128)
    g2b, be2b = vec(_OFF_G2B, 128), vec(_OFF_BE2B, 128)
    lnw2 = _lnt(hat[0:64], g2a[0:64], be2a[0:64])
    lnc2 = _lnt(hat[64:128], g2a[64:128], be2a[64:128])
    lns2 = _lnt(hbt[0:32], g2b[0:32], be2b[0:32])
    lnt2 = _lnt(hbt[32:64], g2b[32:64], be2b[32:64])

    w3a = vec(_OFF_W3A, 128)
    w3b = vec(_OFF_W3B, 128)
    p_w = jnp.sum(lnw2 * w3a[0:64], axis=0, keepdims=True)
    p_c = jnp.sum(lnc2 * w3a[64:128], axis=0, keepdims=True)
    p_s = jnp.sum(lns2 * w3b[0:32], axis=0, keepdims=True)
    p_t = jnp.sum(lnt2 * w3b[32:64], axis=0, keepdims=True)

    lab = lab_ref[...]
    preds = jnp.where(
        lab < 2,
        jnp.where(lab == 0, p_s + vec_ref[_OFF_B3, 0],
                  p_t + vec_ref[_OFF_B3 + 1, 0]),
        jnp.where(lab == 2, p_w + vec_ref[_OFF_B3 + 2, 0],
                  p_c + vec_ref[_OFF_B3 + 3, 0]))
    o_ref[...] = preds


def kernel(x, group_labels, params):
    n, d = x.shape
    blk = 2048
    labels = group_labels.astype(jnp.int32).reshape(1, n)
    pw, pc, ps, pt = (params[k] for k in ("women", "children", "sc", "st"))

    w1 = jnp.concatenate([pw["W1"], pc["W1"], ps["W1"], pt["W1"]], axis=1)
    w2wc = jnp.concatenate([pw["W2"], pc["W2"]], axis=1)
    z = jnp.zeros((64, 64), jnp.float32)
    w2st = jnp.concatenate([ps["W2"], pt["W2"], z], axis=1)
    z64 = jnp.zeros((64,), jnp.float32)
    vecs = jnp.concatenate([
        pw["b1"], pc["b1"], ps["b1"], pt["b1"],
        pw["g1"], pc["g1"], ps["g1"], pt["g1"],
        pw["be1"], pc["be1"], ps["be1"], pt["be1"],
        pw["b2"], pc["b2"], ps["b2"], pt["b2"], z64,
        pw["g2"], pc["g2"], ps["g2"], pt["g2"], z64,
        pw["be2"], pc["be2"], ps["be2"], pt["be2"], z64,
        pw["W3"][:, 0], pc["W3"][:, 0],
        ps["W3"][:, 0], pt["W3"][:, 0], z64,
        ps["b3"], pt["b3"], pw["b3"], pc["b3"],
        jnp.zeros((124,), jnp.float32),
    ])[:, None]

    out = pl.pallas_call(
        _body,
        grid=(n // blk,),
        in_specs=[
            pl.BlockSpec((blk, d), lambda i: (i, 0)),
            pl.BlockSpec((1, blk), lambda i: (0, i)),
            pl.BlockSpec(w1.shape, lambda i: (0, 0)),
            pl.BlockSpec(w2wc.shape, lambda i: (0, 0)),
            pl.BlockSpec(w2st.shape, lambda i: (0, 0)),
            pl.BlockSpec(vecs.shape, lambda i: (0, 0)),
        ],
        out_specs=pl.BlockSpec((1, blk), lambda i: (0, i)),
        out_shape=jax.ShapeDtypeStruct((1, n), x.dtype),
        compiler_params=pltpu.CompilerParams(
            dimension_semantics=("arbitrary",)),
    )(x, labels, w1, w2wc, w2st, vecs)
    return out.reshape(n, 1)


# speedup vs baseline: 2.7738x; 1.0847x over previous
import functools

import jax
import jax.numpy as jnp
from jax.experimental import pallas as pl
from jax.experimental.pallas import tpu as pltpu

EPS = 1e-5

_OFF_B1, _OFF_G1, _OFF_BE1 = 0, 384, 768
_OFF_B2A, _OFF_B2B = 1152, 1280
_OFF_G2A, _OFF_G2B = 1344, 1472
_OFF_BE2A, _OFF_BE2B = 1536, 1664
_OFF_W3A, _OFF_W3B, _OFF_B3 = 1728, 1856, 1920
_VEC_LEN = 1928


def _lnt(h, g, be):
    mu = jnp.mean(h, axis=0, keepdims=True)
    var = jnp.mean(h * h, axis=0, keepdims=True) - mu * mu
    return jnp.maximum((h - mu) * jax.lax.rsqrt(var + EPS) * g + be, 0.0)


def _bf(a):
    return a.astype(jnp.bfloat16)


def _body(x_ref, lab_ref, w1_ref, w2wc_ref, w2st_ref, vec_ref, o_ref):
    vec = lambda off, ln: vec_ref[pl.ds(off, ln), :]
    xb = _bf(x_ref[...])

    ht = jax.lax.dot_general(
        _bf(w1_ref[...]), xb, (((0,), (1,)), ((), ())),
        preferred_element_type=jnp.float32)
    ht = ht + vec(_OFF_B1, 384)
    g1 = vec(_OFF_G1, 384)
    be1 = vec(_OFF_BE1, 384)
    ln0 = _lnt(ht[0:128], g1[0:128], be1[0:128])
    ln1 = _lnt(ht[128:256], g1[128:256], be1[128:256])
    lns = _lnt(ht[256:320], g1[256:320], be1[256:320])
    lnt_ = _lnt(ht[320:384], g1[320:384], be1[320:384])

    w2wc = w2wc_ref[...]
    wlanes = jax.lax.broadcasted_iota(jnp.int32, w2wc.shape, 1)
    w2a = jnp.concatenate([jnp.where(wlanes < 64, w2wc, 0.0),
                           jnp.where(wlanes >= 64, w2wc, 0.0)], axis=0)
    ln01 = jnp.concatenate([ln0, ln1], axis=0)
    hat = jax.lax.dot_general(
        _bf(w2a), _bf(ln01), (((0,), (0,)), ((), ())),
        preferred_element_type=jnp.float32)
    hat = hat + vec(_OFF_B2A, 128)

    w2st = w2st_ref[...]
    slanes = jax.lax.broadcasted_iota(jnp.int32, w2st.shape, 1)
    w2b = jnp.concatenate([jnp.where(slanes < 32, w2st, 0.0),
                           jnp.where((slanes >= 32) & (slanes < 64), w2st, 0.0)],
                          axis=0)
    ln23 = jnp.concatenate([lns, lnt_], axis=0)
    hbt = jax.lax.dot_general(
        _bf(w2b), _bf(ln23), (((0,), (0,)), ((), ())),
        preferred_element_type=jnp.float32)
    hbt = hbt[0:64] + vec(_OFF_B2B, 64)

    g2a, be2a = vec(_OFF_G2A, 128), vec(_OFF_BE2A, 128)
    g2b, be2b = vec(_OFF_G2B, 64), vec(_OFF_BE2B, 64)
    lnw2 = _lnt(hat[0:64], g2a[0:64], be2a[0:64])
    lnc2 = _lnt(hat[64:128], g2a[64:128], be2a[64:128])
    lns2 = _lnt(hbt[0:32], g2b[0:32], be2b[0:32])
    lnt2 = _lnt(hbt[32:64], g2b[32:64], be2b[32:64])

    w3a = vec(_OFF_W3A, 128)
    w3b = vec(_OFF_W3B, 64)
    p_w = jnp.sum(lnw2 * w3a[0:64], axis=0, keepdims=True)
    p_c = jnp.sum(lnc2 * w3a[64:128], axis=0, keepdims=True)
    p_s = jnp.sum(lns2 * w3b[0:32], axis=0, keepdims=True)
    p_t = jnp.sum(lnt2 * w3b[32:64], axis=0, keepdims=True)

    lab = lab_ref[...]
    preds = jnp.where(
        lab < 2,
        jnp.where(lab == 0, p_s + vec_ref[_OFF_B3, 0],
                  p_t + vec_ref[_OFF_B3 + 1, 0]),
        jnp.where(lab == 2, p_w + vec_ref[_OFF_B3 + 2, 0],
                  p_c + vec_ref[_OFF_B3 + 3, 0]))
    o_ref[...] = preds


def kernel(x, group_labels, params):
    n, d = x.shape
    blk = 2048
    labels = group_labels.astype(jnp.int32).reshape(1, n)
    pw, pc, ps, pt = (params[k] for k in ("women", "children", "sc", "st"))

    w1 = jnp.concatenate([pw["W1"], pc["W1"], ps["W1"], pt["W1"]], axis=1)
    w2wc = jnp.concatenate([pw["W2"], pc["W2"]], axis=1)
    z = jnp.zeros((64, 64), jnp.float32)
    w2st = jnp.concatenate([ps["W2"], pt["W2"], z], axis=1)
    vecs = jnp.concatenate([
        pw["b1"], pc["b1"], ps["b1"], pt["b1"],
        pw["g1"], pc["g1"], ps["g1"], pt["g1"],
        pw["be1"], pc["be1"], ps["be1"], pt["be1"],
        pw["b2"], pc["b2"], ps["b2"], pt["b2"],
        pw["g2"], pc["g2"], ps["g2"], pt["g2"],
        pw["be2"], pc["be2"], ps["be2"], pt["be2"],
        pw["W3"][:, 0], pc["W3"][:, 0],
        ps["W3"][:, 0], pt["W3"][:, 0],
        ps["b3"], pt["b3"], pw["b3"], pc["b3"],
        jnp.zeros((4,), jnp.float32),
    ])[:, None]
    assert vecs.shape == (_VEC_LEN, 1)

    out = pl.pallas_call(
        _body,
        grid=(n // blk,),
        in_specs=[
            pl.BlockSpec((blk, d), lambda i: (i, 0)),
            pl.BlockSpec((1, blk), lambda i: (0, i)),
            pl.BlockSpec(w1.shape, lambda i: (0, 0)),
            pl.BlockSpec(w2wc.shape, lambda i: (0, 0)),
            pl.BlockSpec(w2st.shape, lambda i: (0, 0)),
            pl.BlockSpec(vecs.shape, lambda i: (0, 0)),
        ],
        out_specs=pl.BlockSpec((1, blk), lambda i: (0, i)),
        out_shape=jax.ShapeDtypeStruct((1, n), x.dtype),
        compiler_params=pltpu.CompilerParams(
            dimension_semantics=("arbitrary",)),
    )(x, labels, w1, w2wc, w2st, vecs)
    return out.reshape(n, 1)
